# baseline (speedup 1.0000x reference)
import functools

import jax
import jax.numpy as jnp
from jax import lax
from jax.experimental import pallas as pl
from jax.experimental.pallas import tpu as pltpu
from jax.experimental.pallas import tpu_sc as plsc

S, C, L, T = 10, 21, 3, 3
B = 16384

NUM_CORES = 2
NUM_SUBCORES = 16
LANES = 16
NW = NUM_CORES * NUM_SUBCORES
BPW = B // NW
NV = BPW // LANES

W4 = S * C * L * T
W3 = S * C * T
W4P = 1920
W3P = 640


def _balancer_kernel(w_hbm, s_hbm, c_hbm, l_hbm, t_hbm,
                     o4_hbm, o3_hbm,
                     w_v, s_v, c_v, l_v, t_v, o4_v, o3_v, sem):
    wid = lax.axis_index("s") * NUM_CORES + lax.axis_index("c")
    base = wid * BPW
    copies = [
        pltpu.async_copy(w_hbm, w_v, sem),
        pltpu.async_copy(s_hbm.at[pl.ds(base, BPW)], s_v, sem),
        pltpu.async_copy(c_hbm.at[pl.ds(base, BPW)], c_v, sem),
        pltpu.async_copy(l_hbm.at[pl.ds(base, BPW)], l_v, sem),
        pltpu.async_copy(t_hbm.at[pl.ds(base, BPW)], t_v, sem),
    ]
    for cp in copies:
        cp.wait()

    def body(i, carry):
        for u in range(4):
            sl = pl.ds((i * 4 + u) * LANES, LANES)
            s = s_v[sl]
            c = c_v[sl]
            l = l_v[sl]
            t = t_v[sl]
            sc = s * C + c
            i4 = (sc * L + l) * T + t
            i3 = sc * T + t + W4
            o4_v[sl] = plsc.load_gather(w_v, [i4])
            o3_v[sl] = plsc.load_gather(w_v, [i3])
        return carry

    lax.fori_loop(0, NV // 4, body, 0)

    out_copies = [
        pltpu.async_copy(o4_v, o4_hbm.at[pl.ds(base, BPW)], sem),
        pltpu.async_copy(o3_v, o3_hbm.at[pl.ds(base, BPW)], sem),
    ]
    for cp in out_copies:
        cp.wait()


@jax.jit
def kernel(weights_sclt, weights_sct, sources, counts, labels, variant_types):
    w = jnp.concatenate([weights_sclt.reshape(-1), weights_sct.reshape(-1)])
    s = sources.astype(jnp.int32)
    c = counts.astype(jnp.int32)
    l = labels.astype(jnp.int32)
    t = variant_types.astype(jnp.int32)

    mesh = plsc.VectorSubcoreMesh(core_axis_name="c", subcore_axis_name="s")
    run = pl.kernel(
        _balancer_kernel, mesh=mesh,
        compiler_params=pltpu.CompilerParams(needs_layout_passes=False),
        out_type=[jax.ShapeDtypeStruct((B,), jnp.float32),
                  jax.ShapeDtypeStruct((B,), jnp.float32)],
        scratch_types=[
            pltpu.VMEM((W4 + W3,), jnp.float32),
            pltpu.VMEM((BPW,), jnp.int32),
            pltpu.VMEM((BPW,), jnp.int32),
            pltpu.VMEM((BPW,), jnp.int32),
            pltpu.VMEM((BPW,), jnp.int32),
            pltpu.VMEM((BPW,), jnp.float32),
            pltpu.VMEM((BPW,), jnp.float32),
            pltpu.SemaphoreType.DMA,
        ],
    )
    w_label, w_source = run(w, s, c, l, t)
    return (w_label, w_source)


# speedup vs baseline: 1.0117x; 1.0117x over previous
import functools

import jax
import jax.numpy as jnp
from jax import lax
from jax.experimental import pallas as pl
from jax.experimental.pallas import tpu as pltpu
from jax.experimental.pallas import tpu_sc as plsc

S, C, L, T = 10, 21, 3, 3
B = 16384

NUM_CORES = 2
NUM_SUBCORES = 16
LANES = 16
NW = NUM_CORES * NUM_SUBCORES
BPW = B // NW
NV = BPW // LANES

W4 = S * C * L * T
W3 = S * C * T
W4P = 1920
W3P = 640


def _balancer_kernel(w_hbm, s_hbm, c_hbm, l_hbm, t_hbm,
                     o4_hbm, o3_hbm,
                     w_v, s_v, c_v, l_v, t_v, o4_v, o3_v, sem):
    wid = lax.axis_index("s") * NUM_CORES + lax.axis_index("c")
    base = wid * BPW
    copies = [
        pltpu.async_copy(w_hbm, w_v, sem),
        pltpu.async_copy(s_hbm.at[pl.ds(base, BPW)], s_v, sem),
        pltpu.async_copy(c_hbm.at[pl.ds(base, BPW)], c_v, sem),
        pltpu.async_copy(l_hbm.at[pl.ds(base, BPW)], l_v, sem),
        pltpu.async_copy(t_hbm.at[pl.ds(base, BPW)], t_v, sem),
    ]
    for cp in copies:
        cp.wait()

    @plsc.parallel_loop(0, NV, 1, unroll=2)
    def body(i):
        sl = pl.ds(i * LANES, LANES)
        s = s_v[sl]
        c = c_v[sl]
        l = l_v[sl]
        t = t_v[sl]
        sc = s * C + c
        i4 = (sc * L + l) * T + t
        i3 = sc * T + t + W4
        o4_v[sl] = plsc.load_gather(w_v, [i4])
        o3_v[sl] = plsc.load_gather(w_v, [i3])

    out_copies = [
        pltpu.async_copy(o4_v, o4_hbm.at[pl.ds(base, BPW)], sem),
        pltpu.async_copy(o3_v, o3_hbm.at[pl.ds(base, BPW)], sem),
    ]
    for cp in out_copies:
        cp.wait()


@jax.jit
def kernel(weights_sclt, weights_sct, sources, counts, labels, variant_types):
    w = jnp.concatenate([weights_sclt.reshape(-1), weights_sct.reshape(-1)])
    s = sources.astype(jnp.int32)
    c = counts.astype(jnp.int32)
    l = labels.astype(jnp.int32)
    t = variant_types.astype(jnp.int32)

    mesh = plsc.VectorSubcoreMesh(core_axis_name="c", subcore_axis_name="s")
    run = pl.kernel(
        _balancer_kernel, mesh=mesh,
        compiler_params=pltpu.CompilerParams(needs_layout_passes=False),
        out_type=[jax.ShapeDtypeStruct((B,), jnp.float32),
                  jax.ShapeDtypeStruct((B,), jnp.float32)],
        scratch_types=[
            pltpu.VMEM((W4 + W3,), jnp.float32),
            pltpu.VMEM((BPW,), jnp.int32),
            pltpu.VMEM((BPW,), jnp.int32),
            pltpu.VMEM((BPW,), jnp.int32),
            pltpu.VMEM((BPW,), jnp.int32),
            pltpu.VMEM((BPW,), jnp.float32),
            pltpu.VMEM((BPW,), jnp.float32),
            pltpu.SemaphoreType.DMA,
        ],
    )
    w_label, w_source = run(w, s, c, l, t)
    return (w_label, w_source)


# speedup vs baseline: 1.0138x; 1.0021x over previous
import functools

import jax
import jax.numpy as jnp
from jax import lax
from jax.experimental import pallas as pl
from jax.experimental.pallas import tpu as pltpu
from jax.experimental.pallas import tpu_sc as plsc

S, C, L, T = 10, 21, 3, 3
B = 16384

NUM_CORES = 2
NUM_SUBCORES = 16
LANES = 16
NW = NUM_CORES * NUM_SUBCORES
BPW = B // NW
NV = BPW // LANES

W4 = S * C * L * T
W3 = S * C * T
W4P = 1920
W3P = 640


def _balancer_kernel(w_hbm, s_hbm, c_hbm, l_hbm, t_hbm,
                     o4_hbm, o3_hbm,
                     w_v, s_v, c_v, l_v, t_v, o4_v, o3_v, sem):
    wid = lax.axis_index("s") * NUM_CORES + lax.axis_index("c")
    base = wid * BPW
    copies = [
        pltpu.async_copy(w_hbm, w_v, sem),
        pltpu.async_copy(s_hbm.at[pl.ds(base, BPW)], s_v, sem),
        pltpu.async_copy(c_hbm.at[pl.ds(base, BPW)], c_v, sem),
        pltpu.async_copy(l_hbm.at[pl.ds(base, BPW)], l_v, sem),
        pltpu.async_copy(t_hbm.at[pl.ds(base, BPW)], t_v, sem),
    ]
    for cp in copies:
        cp.wait()

    @plsc.parallel_loop(0, NV, 1, unroll=4)
    def body(i):
        sl = pl.ds(i * LANES, LANES)
        s = s_v[sl]
        c = c_v[sl]
        l = l_v[sl]
        t = t_v[sl]
        sc = s * C + c
        i4 = (sc * L + l) * T + t
        i3 = sc * T + t + W4
        o4_v[sl] = plsc.load_gather(w_v, [i4])
        o3_v[sl] = plsc.load_gather(w_v, [i3])

    out_copies = [
        pltpu.async_copy(o4_v, o4_hbm.at[pl.ds(base, BPW)], sem),
        pltpu.async_copy(o3_v, o3_hbm.at[pl.ds(base, BPW)], sem),
    ]
    for cp in out_copies:
        cp.wait()


@jax.jit
def kernel(weights_sclt, weights_sct, sources, counts, labels, variant_types):
    w = jnp.concatenate([weights_sclt.reshape(-1), weights_sct.reshape(-1)])
    s = sources.astype(jnp.int32)
    c = counts.astype(jnp.int32)
    l = labels.astype(jnp.int32)
    t = variant_types.astype(jnp.int32)

    mesh = plsc.VectorSubcoreMesh(core_axis_name="c", subcore_axis_name="s")
    run = pl.kernel(
        _balancer_kernel, mesh=mesh,
        compiler_params=pltpu.CompilerParams(needs_layout_passes=False),
        out_type=[jax.ShapeDtypeStruct((B,), jnp.float32),
                  jax.ShapeDtypeStruct((B,), jnp.float32)],
        scratch_types=[
            pltpu.VMEM((W4 + W3,), jnp.float32),
            pltpu.VMEM((BPW,), jnp.int32),
            pltpu.VMEM((BPW,), jnp.int32),
            pltpu.VMEM((BPW,), jnp.int32),
            pltpu.VMEM((BPW,), jnp.int32),
            pltpu.VMEM((BPW,), jnp.float32),
            pltpu.VMEM((BPW,), jnp.float32),
            pltpu.SemaphoreType.DMA,
        ],
    )
    w_label, w_source = run(w, s, c, l, t)
    return (w_label, w_source)
